# Initial kernel scaffold; baseline (speedup 1.0000x reference)
import jax
import jax.numpy as jnp
from jax.experimental import pallas as pl


def kernel(features, edge_index, W1, b1, W2, b2):
    raise NotImplementedError("write your pallas kernel here")


# speedup vs baseline: 1.6218x; 1.6218x over previous
import jax
import jax.numpy as jnp
from jax.experimental import pallas as pl


def _bias_kernel(x_ref, b_ref, o_ref):
    o_ref[...] = x_ref[...] + b_ref[...]


def _gcn_layer_nobias(x, W, norm_src, norm_dst, src, dst):
    xw = (x @ W) * norm_src[:, None]
    msg = jnp.take(xw, src, axis=0)
    agg = jnp.zeros_like(xw).at[dst].add(msg)
    return agg * norm_dst[:, None]


def kernel(features, edge_index, W1, b1, W2, b2):
    src = edge_index[0]
    dst = edge_index[1]
    n = features.shape[0]
    ones = jnp.ones((src.shape[0],), dtype=features.dtype)
    out_deg = jnp.zeros((n,), dtype=features.dtype).at[src].add(ones)
    in_deg = jnp.zeros((n,), dtype=features.dtype).at[dst].add(ones)
    norm_src = jax.lax.rsqrt(jnp.clip(out_deg, 1.0, None))
    norm_dst = jax.lax.rsqrt(jnp.clip(in_deg, 1.0, None))

    def bias_add(x, b):
        return pl.pallas_call(
            _bias_kernel,
            out_shape=jax.ShapeDtypeStruct(x.shape, x.dtype),
        )(x, jnp.broadcast_to(b, x.shape))

    h = jax.nn.relu(bias_add(_gcn_layer_nobias(features, W1, norm_src, norm_dst, src, dst), b1))
    out = bias_add(_gcn_layer_nobias(h, W2, norm_src, norm_dst, src, dst), b2)
    return out


# speedup vs baseline: 7.7902x; 4.8036x over previous
import functools

import jax
import jax.numpy as jnp
from jax import lax
from jax.experimental import pallas as pl
from jax.experimental.pallas import tpu as pltpu
from jax.experimental.pallas import tpu_sc as plsc

_N = 10000
_D = 128
_E = 320000
_NC = 2
_NS = 16
_NW = _NC * _NS
_EPW = _E // _NW
_CHUNK = 80
_NCHUNK = _EPW // _CHUNK
_ZR = 80
_NZCH = _N // _ZR
_CPT = (_NZCH + _NS - 1) // _NS

_mesh = plsc.VectorSubcoreMesh(core_axis_name="c", subcore_axis_name="s")


def _deg_body(src_h, dst_h, outdeg, indeg, ones_v, zb, sidx, didx, acc_o, acc_i):
    c = lax.axis_index("c")
    s = lax.axis_index("s")
    base = (c * _NS + s) * _EPW

    @pl.loop(0, _CHUNK)
    def _(r):
        ones_v[r, :] = jnp.ones((16,), jnp.float32)

    @pl.loop(0, _ZR)
    def _(r):
        zb[r, :] = jnp.zeros((16,), jnp.float32)

    @pl.loop(0, _CPT)
    def _(i):
        k = s + i * _NS

        @pl.when(k < _NZCH)
        def _():
            rows = pl.ds(pl.multiple_of(k * _ZR, 8), _ZR)
            pltpu.sync_copy(zb, acc_o.at[rows])
            pltpu.sync_copy(zb, acc_i.at[rows])

    plsc.subcore_barrier()

    @pl.loop(0, _NCHUNK)
    def _(j):
        e = pl.ds(base + j * _CHUNK, _CHUNK)
        pltpu.sync_copy(src_h.at[e], sidx)
        pltpu.sync_copy(dst_h.at[e], didx)
        pltpu.sync_copy(ones_v, acc_o.at[sidx], add=True)
        pltpu.sync_copy(ones_v, acc_i.at[didx], add=True)

    plsc.subcore_barrier()

    @pl.loop(0, _CPT)
    def _(i):
        k = s + i * _NS

        @pl.when(k < _NZCH)
        def _():
            rows = pl.ds(pl.multiple_of(k * _ZR, 8), _ZR)
            pltpu.sync_copy(acc_o.at[rows], zb)
            pltpu.sync_copy(zb, outdeg.at[c, rows])
            pltpu.sync_copy(acc_i.at[rows], zb)
            pltpu.sync_copy(zb, indeg.at[c, rows])


_sc_deg = pl.kernel(
    _deg_body,
    out_type=[
        jax.ShapeDtypeStruct((_NC, _N, 16), jnp.float32),
        jax.ShapeDtypeStruct((_NC, _N, 16), jnp.float32),
    ],
    mesh=_mesh,
    scratch_types=[
        pltpu.VMEM((_CHUNK, 16), jnp.float32),
        pltpu.VMEM((_ZR, 16), jnp.float32),
        pltpu.VMEM((_CHUNK,), jnp.int32),
        pltpu.VMEM((_CHUNK,), jnp.int32),
        pltpu.VMEM_SHARED((_N, 16), jnp.float32),
        pltpu.VMEM_SHARED((_N, 16), jnp.float32),
    ],
)


def _agg_body(xs, src_h, dst_h, part, sidx, didx, buf, zb, acc, sem):
    c = lax.axis_index("c")
    s = lax.axis_index("s")
    base = (c * _NS + s) * _EPW

    @pl.loop(0, _ZR)
    def _(r):
        @pl.loop(0, _D, step=16)
        def _(k):
            zb[r, pl.ds(k, 16)] = jnp.zeros((16,), jnp.float32)

    @pl.loop(0, _CPT)
    def _(i):
        k = s + i * _NS

        @pl.when(k < _NZCH)
        def _():
            pltpu.sync_copy(zb, acc.at[pl.ds(pl.multiple_of(k * _ZR, 8), _ZR)])

    plsc.subcore_barrier()

    @pl.loop(0, _NCHUNK)
    def _(j):
        e = pl.ds(base + j * _CHUNK, _CHUNK)
        pltpu.sync_copy(src_h.at[e], sidx)
        pltpu.async_copy(xs.at[sidx], buf, sem).wait()
        pltpu.sync_copy(dst_h.at[e], didx)
        pltpu.sync_copy(buf, acc.at[didx], add=True)

    plsc.subcore_barrier()

    @pl.loop(0, _CPT)
    def _(i):
        k = s + i * _NS

        @pl.when(k < _NZCH)
        def _():
            rows = pl.ds(pl.multiple_of(k * _ZR, 8), _ZR)
            pltpu.sync_copy(acc.at[rows], zb)
            pltpu.sync_copy(zb, part.at[c, rows])


_sc_agg = pl.kernel(
    _agg_body,
    out_type=jax.ShapeDtypeStruct((_NC, _N, _D), jnp.float32),
    mesh=_mesh,
    scratch_types=[
        pltpu.VMEM((_CHUNK,), jnp.int32),
        pltpu.VMEM((_CHUNK,), jnp.int32),
        pltpu.VMEM((_CHUNK, _D), jnp.float32),
        pltpu.VMEM((_ZR, _D), jnp.float32),
        pltpu.VMEM_SHARED((_N, _D), jnp.float32),
        pltpu.SemaphoreType.DMA,
    ],
)

_ROWS = 1000


def _mm_body(x_ref, w_ref, o_ref):
    o_ref[...] = jnp.dot(x_ref[...], w_ref[...], preferred_element_type=jnp.float32)


def _tc_matmul(x, W):
    return pl.pallas_call(
        _mm_body,
        grid=(_N // _ROWS,),
        in_specs=[
            pl.BlockSpec((_ROWS, _D), lambda i: (i, 0)),
            pl.BlockSpec((_D, _D), lambda i: (0, 0)),
        ],
        out_specs=pl.BlockSpec((_ROWS, _D), lambda i: (i, 0)),
        out_shape=jax.ShapeDtypeStruct((_N, _D), jnp.float32),
    )(x, W)


def _norm_from(pd_ref):
    deg = pd_ref[0] + pd_ref[1]
    return lax.rsqrt(jnp.clip(deg, 1.0, None))[:, 0:1]


def _scale_body(x_ref, pdo_ref, o_ref):
    o_ref[...] = x_ref[...] * _norm_from(pdo_ref)


def _tc_scale(x, pdo):
    return pl.pallas_call(
        _scale_body,
        grid=(_N // _ROWS,),
        in_specs=[
            pl.BlockSpec((_ROWS, _D), lambda i: (i, 0)),
            pl.BlockSpec((_NC, _ROWS, 16), lambda i: (0, i, 0)),
        ],
        out_specs=pl.BlockSpec((_ROWS, _D), lambda i: (i, 0)),
        out_shape=jax.ShapeDtypeStruct((_N, _D), jnp.float32),
    )(x, pdo)


def _layer2_body(p_ref, pdi_ref, pdo_ref, b1_ref, w_ref, o_ref):
    nd = _norm_from(pdi_ref)
    ns = _norm_from(pdo_ref)
    agg = p_ref[0] + p_ref[1]
    h = jnp.maximum(agg * nd + b1_ref[...], 0.0)
    o_ref[...] = jnp.dot(h, w_ref[...], preferred_element_type=jnp.float32) * ns


def _tc_layer2(part, pdi, pdo, b1, W2):
    return pl.pallas_call(
        _layer2_body,
        grid=(_N // _ROWS,),
        in_specs=[
            pl.BlockSpec((_NC, _ROWS, _D), lambda i: (0, i, 0)),
            pl.BlockSpec((_NC, _ROWS, 16), lambda i: (0, i, 0)),
            pl.BlockSpec((_NC, _ROWS, 16), lambda i: (0, i, 0)),
            pl.BlockSpec((1, _D), lambda i: (0, 0)),
            pl.BlockSpec((_D, _D), lambda i: (0, 0)),
        ],
        out_specs=pl.BlockSpec((_ROWS, _D), lambda i: (i, 0)),
        out_shape=jax.ShapeDtypeStruct((_N, _D), jnp.float32),
    )(part, pdi, pdo, b1, W2)


def _final_body(p_ref, pdi_ref, b2_ref, o_ref):
    nd = _norm_from(pdi_ref)
    o_ref[...] = (p_ref[0] + p_ref[1]) * nd + b2_ref[...]


def _tc_final(part, pdi, b2):
    return pl.pallas_call(
        _final_body,
        grid=(_N // _ROWS,),
        in_specs=[
            pl.BlockSpec((_NC, _ROWS, _D), lambda i: (0, i, 0)),
            pl.BlockSpec((_NC, _ROWS, 16), lambda i: (0, i, 0)),
            pl.BlockSpec((1, _D), lambda i: (0, 0)),
        ],
        out_specs=pl.BlockSpec((_ROWS, _D), lambda i: (i, 0)),
        out_shape=jax.ShapeDtypeStruct((_N, _D), jnp.float32),
    )(part, pdi, b2)


def kernel(features, edge_index, W1, b1, W2, b2):
    src = edge_index[0].astype(jnp.int32)
    dst = edge_index[1].astype(jnp.int32)
    pdo, pdi = _sc_deg(src, dst)
    xw1 = _tc_matmul(features, W1)
    xs1 = _tc_scale(xw1, pdo)
    part1 = _sc_agg(xs1, src, dst)
    xs2 = _tc_layer2(part1, pdi, pdo, b1.reshape(1, _D), W2)
    part2 = _sc_agg(xs2, src, dst)
    out = _tc_final(part2, pdi, b2.reshape(1, _D))
    return out


# speedup vs baseline: 8.8021x; 1.1299x over previous
import jax
import jax.numpy as jnp
from jax import lax
from jax.experimental import pallas as pl
from jax.experimental.pallas import tpu as pltpu
from jax.experimental.pallas import tpu_sc as plsc

_N = 10000
_D = 128
_E = 320000
_NC = 2
_NS = 16
_NW = _NC * _NS
_EPW = _E // _NW
_CHUNK = 40
_NCHUNK = _EPW // _CHUNK
_ZR = 40
_NZCH = _N // _ZR
_CPT = (_NZCH + _NS - 1) // _NS

_mesh = plsc.VectorSubcoreMesh(core_axis_name="c", subcore_axis_name="s")


def _idx_start(src_h, dst_h, j, sidx, didx, sem):
    e = pl.ds(j * _CHUNK, _CHUNK)
    pltpu.async_copy(src_h.at[e], sidx, sem)
    pltpu.async_copy(dst_h.at[e], didx, sem)


def _idx_wait(src_h, dst_h, j, sidx, didx, sem):
    e = pl.ds(j * _CHUNK, _CHUNK)
    pltpu.make_async_copy(src_h.at[e], sidx, sem).wait()
    pltpu.make_async_copy(dst_h.at[e], didx, sem).wait()


def _deg_body(src_h, dst_h, outdeg, indeg, ones_v, zb,
              sidx0, didx0, sidx1, didx1, acc_o, acc_i,
              isem0, isem1, osem, psem):
    c = lax.axis_index("c")
    s = lax.axis_index("s")
    w = c * _NS + s
    base = w * _NCHUNK

    @pl.loop(0, _CHUNK)
    def _(r):
        ones_v[r, :] = jnp.ones((16,), jnp.float32)
        zb[r, :] = jnp.zeros((16,), jnp.float32)

    @pl.loop(0, _CPT)
    def _(i):
        k = s + i * _NS

        @pl.when(k < _NZCH)
        def _():
            rows = pl.ds(pl.multiple_of(k * _ZR, 8), _ZR)
            pltpu.sync_copy(zb, acc_o.at[rows])
            pltpu.sync_copy(zb, acc_i.at[rows])

    _idx_start(src_h, dst_h, base, sidx0, didx0, isem0)
    _idx_start(src_h, dst_h, base + 1, sidx1, didx1, isem1)

    plsc.subcore_barrier()

    def half(j, sidx, didx, isem):
        _idx_wait(src_h, dst_h, base + j, sidx, didx, isem)
        a = pltpu.async_copy(ones_v, acc_o.at[sidx], osem, add=True)
        b = pltpu.async_copy(ones_v, acc_i.at[didx], psem, add=True)
        a.wait()
        b.wait()

        @pl.when(j + 2 < _NCHUNK)
        def _():
            _idx_start(src_h, dst_h, base + j + 2, sidx, didx, isem)

    @pl.loop(0, _NCHUNK // 2)
    def _(k):
        half(k * 2, sidx0, didx0, isem0)
        half(k * 2 + 1, sidx1, didx1, isem1)

    plsc.subcore_barrier()

    @pl.loop(0, _CPT)
    def _(i):
        k = s + i * _NS

        @pl.when(k < _NZCH)
        def _():
            rows = pl.ds(pl.multiple_of(k * _ZR, 8), _ZR)
            pltpu.sync_copy(acc_o.at[rows], zb)
            pltpu.sync_copy(zb, outdeg.at[c, rows])
            pltpu.sync_copy(acc_i.at[rows], zb)
            pltpu.sync_copy(zb, indeg.at[c, rows])


_sc_deg = pl.kernel(
    _deg_body,
    out_type=[
        jax.ShapeDtypeStruct((_NC, _N, 16), jnp.float32),
        jax.ShapeDtypeStruct((_NC, _N, 16), jnp.float32),
    ],
    mesh=_mesh,
    scratch_types=[
        pltpu.VMEM((_CHUNK, 16), jnp.float32),
        pltpu.VMEM((_CHUNK, 16), jnp.float32),
        pltpu.VMEM((_CHUNK,), jnp.int32),
        pltpu.VMEM((_CHUNK,), jnp.int32),
        pltpu.VMEM((_CHUNK,), jnp.int32),
        pltpu.VMEM((_CHUNK,), jnp.int32),
        pltpu.VMEM_SHARED((_N, 16), jnp.float32),
        pltpu.VMEM_SHARED((_N, 16), jnp.float32),
        pltpu.SemaphoreType.DMA,
        pltpu.SemaphoreType.DMA,
        pltpu.SemaphoreType.DMA,
        pltpu.SemaphoreType.DMA,
    ],
)


def _agg_body(xs, src_h, dst_h, part,
              sidx0, didx0, sidx1, didx1, buf0, buf1, acc,
              isem0, isem1, gsem0, gsem1):
    c = lax.axis_index("c")
    s = lax.axis_index("s")
    w = c * _NS + s
    base = w * _NCHUNK

    @pl.loop(0, _ZR)
    def _(r):
        @pl.loop(0, _D, step=16)
        def _(q):
            buf0[r, pl.ds(q, 16)] = jnp.zeros((16,), jnp.float32)

    @pl.loop(0, _CPT)
    def _(i):
        k = s + i * _NS

        @pl.when(k < _NZCH)
        def _():
            pltpu.sync_copy(buf0, acc.at[pl.ds(pl.multiple_of(k * _ZR, 8), _ZR)])

    _idx_start(src_h, dst_h, base, sidx0, didx0, isem0)
    _idx_start(src_h, dst_h, base + 1, sidx1, didx1, isem1)

    plsc.subcore_barrier()

    def gat_start(sidx, buf, sem):
        pltpu.async_copy(xs.at[sidx], buf, sem)

    def gat_wait(sidx, buf, sem):
        pltpu.make_async_copy(xs.at[sidx], buf, sem).wait()

    _idx_wait(src_h, dst_h, base, sidx0, didx0, isem0)
    gat_start(sidx0, buf0, gsem0)

    def half(j, sidx, didx, buf, isem, gsem, sidx_n, didx_n, buf_n, isem_n, gsem_n):
        gat_wait(sidx, buf, gsem)
        pltpu.sync_copy(buf, acc.at[didx], add=True)

        @pl.when(j + 2 < _NCHUNK)
        def _():
            _idx_start(src_h, dst_h, base + j + 2, sidx, didx, isem)

        @pl.when(j + 1 < _NCHUNK)
        def _():
            _idx_wait(src_h, dst_h, base + j + 1, sidx_n, didx_n, isem_n)
            gat_start(sidx_n, buf_n, gsem_n)

    @pl.loop(0, _NCHUNK // 2)
    def _(k):
        half(k * 2, sidx0, didx0, buf0, isem0, gsem0,
             sidx1, didx1, buf1, isem1, gsem1)
        half(k * 2 + 1, sidx1, didx1, buf1, isem1, gsem1,
             sidx0, didx0, buf0, isem0, gsem0)

    plsc.subcore_barrier()

    @pl.loop(0, _CPT)
    def _(i):
        k = s + i * _NS

        @pl.when(k < _NZCH)
        def _():
            rows = pl.ds(pl.multiple_of(k * _ZR, 8), _ZR)
            pltpu.sync_copy(acc.at[rows], buf0)
            pltpu.sync_copy(buf0, part.at[c, rows])


_sc_agg = pl.kernel(
    _agg_body,
    out_type=jax.ShapeDtypeStruct((_NC, _N, _D), jnp.float32),
    mesh=_mesh,
    scratch_types=[
        pltpu.VMEM((_CHUNK,), jnp.int32),
        pltpu.VMEM((_CHUNK,), jnp.int32),
        pltpu.VMEM((_CHUNK,), jnp.int32),
        pltpu.VMEM((_CHUNK,), jnp.int32),
        pltpu.VMEM((_CHUNK, _D), jnp.float32),
        pltpu.VMEM((_CHUNK, _D), jnp.float32),
        pltpu.VMEM_SHARED((_N, _D), jnp.float32),
        pltpu.SemaphoreType.DMA,
        pltpu.SemaphoreType.DMA,
        pltpu.SemaphoreType.DMA,
        pltpu.SemaphoreType.DMA,
    ],
)

_ROWS = 1000


def _mm_body(x_ref, w_ref, o_ref):
    o_ref[...] = jnp.dot(x_ref[...], w_ref[...], preferred_element_type=jnp.float32)


def _tc_matmul(x, W):
    return pl.pallas_call(
        _mm_body,
        grid=(_N // _ROWS,),
        in_specs=[
            pl.BlockSpec((_ROWS, _D), lambda i: (i, 0)),
            pl.BlockSpec((_D, _D), lambda i: (0, 0)),
        ],
        out_specs=pl.BlockSpec((_ROWS, _D), lambda i: (i, 0)),
        out_shape=jax.ShapeDtypeStruct((_N, _D), jnp.float32),
    )(x, W)


def _norm_from(pd_ref):
    deg = pd_ref[0] + pd_ref[1]
    return lax.rsqrt(jnp.clip(deg, 1.0, None))[:, 0:1]


def _scale_body(x_ref, pdo_ref, o_ref):
    o_ref[...] = x_ref[...] * _norm_from(pdo_ref)


def _tc_scale(x, pdo):
    return pl.pallas_call(
        _scale_body,
        grid=(_N // _ROWS,),
        in_specs=[
            pl.BlockSpec((_ROWS, _D), lambda i: (i, 0)),
            pl.BlockSpec((_NC, _ROWS, 16), lambda i: (0, i, 0)),
        ],
        out_specs=pl.BlockSpec((_ROWS, _D), lambda i: (i, 0)),
        out_shape=jax.ShapeDtypeStruct((_N, _D), jnp.float32),
    )(x, pdo)


def _layer2_body(p_ref, pdi_ref, pdo_ref, b1_ref, w_ref, o_ref):
    nd = _norm_from(pdi_ref)
    ns = _norm_from(pdo_ref)
    agg = p_ref[0] + p_ref[1]
    h = jnp.maximum(agg * nd + b1_ref[...], 0.0)
    o_ref[...] = jnp.dot(h, w_ref[...], preferred_element_type=jnp.float32) * ns


def _tc_layer2(part, pdi, pdo, b1, W2):
    return pl.pallas_call(
        _layer2_body,
        grid=(_N // _ROWS,),
        in_specs=[
            pl.BlockSpec((_NC, _ROWS, _D), lambda i: (0, i, 0)),
            pl.BlockSpec((_NC, _ROWS, 16), lambda i: (0, i, 0)),
            pl.BlockSpec((_NC, _ROWS, 16), lambda i: (0, i, 0)),
            pl.BlockSpec((1, _D), lambda i: (0, 0)),
            pl.BlockSpec((_D, _D), lambda i: (0, 0)),
        ],
        out_specs=pl.BlockSpec((_ROWS, _D), lambda i: (i, 0)),
        out_shape=jax.ShapeDtypeStruct((_N, _D), jnp.float32),
    )(part, pdi, pdo, b1, W2)


def _final_body(p_ref, pdi_ref, b2_ref, o_ref):
    nd = _norm_from(pdi_ref)
    o_ref[...] = (p_ref[0] + p_ref[1]) * nd + b2_ref[...]


def _tc_final(part, pdi, b2):
    return pl.pallas_call(
        _final_body,
        grid=(_N // _ROWS,),
        in_specs=[
            pl.BlockSpec((_NC, _ROWS, _D), lambda i: (0, i, 0)),
            pl.BlockSpec((_NC, _ROWS, 16), lambda i: (0, i, 0)),
            pl.BlockSpec((1, _D), lambda i: (0, 0)),
        ],
        out_specs=pl.BlockSpec((_ROWS, _D), lambda i: (i, 0)),
        out_shape=jax.ShapeDtypeStruct((_N, _D), jnp.float32),
    )(part, pdi, b2)


def kernel(features, edge_index, W1, b1, W2, b2):
    src = edge_index[0].astype(jnp.int32)
    dst = edge_index[1].astype(jnp.int32)
    pdo, pdi = _sc_deg(src, dst)
    xw1 = _tc_matmul(features, W1)
    xs1 = _tc_scale(xw1, pdo)
    part1 = _sc_agg(xs1, src, dst)
    xs2 = _tc_layer2(part1, pdi, pdo, b1.reshape(1, _D), W2)
    part2 = _sc_agg(xs2, src, dst)
    out = _tc_final(part2, pdi, b2.reshape(1, _D))
    return out
